# baseline (speedup 1.0000x reference)
import jax
import jax.numpy as jnp
from jax import lax
from jax.experimental import pallas as pl

DTIME_MAX = 10.0


def _body(u_ref, r_ref, t_ref, e_ref, o_ref):
    u = u_ref[...]
    r = r_ref[...]
    t = t_ref[...]
    ex = e_ref[...]
    crit = u * r / t
    m = crit < 1.0
    rb, e_dim = u.shape
    iota = lax.broadcasted_iota(jnp.int32, (rb, e_dim), 1)
    key = jnp.where(m, iota, e_dim)
    kmin = jnp.min(key, axis=1, keepdims=True)
    val = jnp.sum(jnp.where((iota == kmin) & m, ex, 0.0), axis=1)
    o_ref[...] = jnp.where(kmin[:, 0] >= e_dim, DTIME_MAX, val)


def kernel(unif_numbers, sample_rate, total_intensities, exp_numbers):
    B, L, S, E = unif_numbers.shape
    N = B * L * S
    RB = 512 if N % 512 == 0 else N
    u2 = unif_numbers.reshape(N, E)
    t2 = total_intensities.reshape(N, E)
    e2 = exp_numbers.reshape(N, E)
    r2 = jnp.broadcast_to(sample_rate.reshape(B * L, 1), (B * L, S)).reshape(N, 1)

    out = pl.pallas_call(
        _body,
        grid=(N // RB,),
        in_specs=[
            pl.BlockSpec((RB, E), lambda i: (i, 0)),
            pl.BlockSpec((RB, 1), lambda i: (i, 0)),
            pl.BlockSpec((RB, E), lambda i: (i, 0)),
            pl.BlockSpec((RB, E), lambda i: (i, 0)),
        ],
        out_specs=pl.BlockSpec((RB,), lambda i: (i,)),
        out_shape=jax.ShapeDtypeStruct((N,), jnp.float32),
    )(u2, r2, t2, e2)
    return out.reshape(B, L, S)


# speedup vs baseline: 1.0345x; 1.0345x over previous
import jax
import jax.numpy as jnp
from jax import lax
from jax.experimental import pallas as pl
from jax.experimental.pallas import tpu as pltpu
from jax.experimental.pallas import tpu_sc as plsc

DTIME_MAX = 10.0

NC = 2
NS = 16
NW = NC * NS
LANES = 16

E_DIM = 100
HEAD = 20
RB = 400
PB = 80
NG = RB // LANES
SENT = HEAD


def _sc_body(uh_hbm, th_hbm, eh_hbm, rate_hbm, u2_hbm, t2_hbm, e2_hbm, out_hbm,
             u_v, t_v, rate_v, km_v, idx_v, val_v,
             u2_v, t2_v, e2_v, cnt_s, sem):
    N = u2_hbm.shape[0]
    RW = N // NW
    NB = RW // RB

    wid = lax.axis_index("s") * NC + lax.axis_index("c")
    base_w = wid * RW

    lane_iota = lax.iota(jnp.int32, LANES)

    def block_body(b, _):
        r0 = base_w + b * RB
        pltpu.sync_copy(rate_hbm.at[pl.ds(r0, RB)], rate_v)
        pltpu.sync_copy(uh_hbm.at[pl.ds(r0 * HEAD, RB * HEAD)], u_v)
        pltpu.sync_copy(th_hbm.at[pl.ds(r0 * HEAD, RB * HEAD)], t_v)
        cnt_s[0] = 0

        def group_body(g, _):
            g16 = g * LANES
            rows_l = g16 + lane_iota
            rate_g = rate_v[pl.ds(g16, LANES)]
            base = rows_l * HEAD
            km = jnp.full((LANES,), SENT, jnp.int32)
            for e in range(HEAD):
                uc = plsc.load_gather(u_v, [base + e])
                tc = plsc.load_gather(t_v, [base + e])
                crit = uc * rate_g / tc
                acc = crit < 1.0
                km = jnp.minimum(
                    km, jnp.where(acc,
                                  jnp.full((LANES,), e, jnp.int32),
                                  jnp.full((LANES,), SENT, jnp.int32)))
            km_v[pl.ds(g16, LANES)] = km
            idx_v[pl.ds(g16, LANES)] = (r0 + rows_l) * HEAD + jnp.minimum(
                km, jnp.full((LANES,), HEAD - 1, jnp.int32))
            cnt_s[0] = cnt_s[0] + jnp.sum((km >= SENT).astype(jnp.int32))
            return _

        lax.fori_loop(0, NG, group_body, None)

        pltpu.async_copy(eh_hbm.at[idx_v], val_v, sem).wait()

        @pl.when(cnt_s[0] > 0)
        def _phase2():
            for sub in range(RB // PB):
                s0 = sub * PB

                pltpu.sync_copy(u2_hbm.at[pl.ds(r0 + s0, PB), :], u2_v)
                pltpu.sync_copy(t2_hbm.at[pl.ds(r0 + s0, PB), :], t2_v)
                pltpu.sync_copy(e2_hbm.at[pl.ds(r0 + s0, PB), :], e2_v)

                def g2_body(g2, _):
                    g16 = g2 * LANES
                    rows16 = g16 + lane_iota
                    km_g = km_v[pl.ds(s0 + g16, LANES)]
                    unres = km_g >= SENT
                    rate_g = rate_v[pl.ds(s0 + g16, LANES)]

                    def col_body(e, km2):
                        col = jnp.broadcast_to(e, (LANES,))
                        uc = plsc.load_gather(u2_v, [rows16, col])
                        tc = plsc.load_gather(t2_v, [rows16, col])
                        crit = uc * rate_g / tc
                        acc = crit < 1.0
                        return jnp.minimum(
                            km2, jnp.where(acc, col,
                                           jnp.full((LANES,), E_DIM, jnp.int32)))

                    km2 = lax.fori_loop(
                        HEAD, E_DIM, col_body,
                        jnp.full((LANES,), E_DIM, jnp.int32))
                    found2 = km2 < E_DIM
                    val2 = plsc.load_gather(
                        e2_v, [rows16,
                               jnp.minimum(km2, jnp.full((LANES,), E_DIM - 1,
                                                         jnp.int32))])
                    vg = val_v[pl.ds(s0 + g16, LANES)]
                    val_v[pl.ds(s0 + g16, LANES)] = jnp.where(
                        unres,
                        jnp.where(found2, val2,
                                  jnp.full((LANES,), DTIME_MAX, jnp.float32)),
                        vg)
                    return _
                lax.fori_loop(0, PB // LANES, g2_body, None)

        pltpu.sync_copy(val_v, out_hbm.at[pl.ds(r0, RB)])
        return _

    lax.fori_loop(0, NB, block_body, None)


def kernel(unif_numbers, sample_rate, total_intensities, exp_numbers):
    B, L, S, E = unif_numbers.shape
    N = B * L * S
    u2 = unif_numbers.reshape(N, E)
    t2 = total_intensities.reshape(N, E)
    e2 = exp_numbers.reshape(N, E)
    uh = u2[:, :HEAD].reshape(N * HEAD)
    th = t2[:, :HEAD].reshape(N * HEAD)
    eh = e2[:, :HEAD].reshape(N * HEAD)
    r1 = jnp.broadcast_to(sample_rate.reshape(B * L, 1), (B * L, S)).reshape(N)

    mesh = plsc.VectorSubcoreMesh(core_axis_name="c", subcore_axis_name="s")
    run = pl.kernel(
        _sc_body,
        out_type=jax.ShapeDtypeStruct((N,), jnp.float32),
        mesh=mesh,
        scratch_types=[
            pltpu.VMEM((RB * HEAD,), jnp.float32),
            pltpu.VMEM((RB * HEAD,), jnp.float32),
            pltpu.VMEM((RB,), jnp.float32),
            pltpu.VMEM((RB,), jnp.int32),
            pltpu.VMEM((RB,), jnp.int32),
            pltpu.VMEM((RB,), jnp.float32),
            pltpu.VMEM((PB, E_DIM), jnp.float32),
            pltpu.VMEM((PB, E_DIM), jnp.float32),
            pltpu.VMEM((PB, E_DIM), jnp.float32),
            pltpu.SMEM((1,), jnp.int32),
            pltpu.SemaphoreType.DMA,
        ],
        compiler_params=pltpu.CompilerParams(needs_layout_passes=False),
    )
    out = run(uh, th, eh, r1, u2, t2, e2)
    return out.reshape(B, L, S)
